# baseline (speedup 1.0000x reference)
import numpy as np
import jax
import jax.numpy as jnp
from jax.experimental import pallas as pl
from jax.experimental.pallas import tpu as pltpu

_MAGIC = 0x1BD11BDA
_R1 = (13, 15, 26, 6)
_R2 = (17, 29, 16, 24)
_M32 = 0xFFFFFFFF


def _tf2x32_py(k1, k2, c1, c2):
    ks0, ks1 = k1, k2
    ks2 = k1 ^ k2 ^ _MAGIC
    x0 = (c1 + ks0) & _M32
    x1 = (c2 + ks1) & _M32

    def four(x0, x1, rs):
        for r in rs:
            x0 = (x0 + x1) & _M32
            x1 = ((x1 << r) | (x1 >> (32 - r))) & _M32
            x1 = x0 ^ x1
        return x0, x1

    x0, x1 = four(x0, x1, _R1); x0 = (x0 + ks1) & _M32; x1 = (x1 + ks2 + 1) & _M32
    x0, x1 = four(x0, x1, _R2); x0 = (x0 + ks2) & _M32; x1 = (x1 + ks0 + 2) & _M32
    x0, x1 = four(x0, x1, _R1); x0 = (x0 + ks0) & _M32; x1 = (x1 + ks1 + 3) & _M32
    x0, x1 = four(x0, x1, _R2); x0 = (x0 + ks1) & _M32; x1 = (x1 + ks2 + 4) & _M32
    x0, x1 = four(x0, x1, _R1); x0 = (x0 + ks2) & _M32; x1 = (x1 + ks0 + 5) & _M32
    return x0, x1


_KA1, _KA2 = _tf2x32_py(0, 42, 0, 0)
_KB1, _KB2 = _tf2x32_py(0, 42, 0, 1)


def _tf2x32(k1, k2, c1, c2):
    sl = jax.lax.shift_left
    sr = jax.lax.shift_right_logical
    ks0, ks1 = k1, k2
    ks2 = k1 ^ k2 ^ np.uint32(_MAGIC)
    x0 = c1 + ks0
    x1 = c2 + ks1

    def four(x0, x1, rs):
        for r in rs:
            x0 = x0 + x1
            x1 = sl(x1, np.uint32(r)) | sr(x1, np.uint32(32 - r))
            x1 = x0 ^ x1
        return x0, x1

    x0, x1 = four(x0, x1, _R1); x0 = x0 + ks1; x1 = x1 + (ks2 + np.uint32(1))
    x0, x1 = four(x0, x1, _R2); x0 = x0 + ks2; x1 = x1 + (ks0 + np.uint32(2))
    x0, x1 = four(x0, x1, _R1); x0 = x0 + ks0; x1 = x1 + (ks1 + np.uint32(3))
    x0, x1 = four(x0, x1, _R2); x0 = x0 + ks1; x1 = x1 + (ks2 + np.uint32(4))
    x0, x1 = four(x0, x1, _R1); x0 = x0 + ks2; x1 = x1 + (ks0 + np.uint32(5))
    return x0, x1


def _bits_to_unit(bits):
    fb = jax.lax.shift_right_logical(bits, np.uint32(9)) | np.uint32(0x3F800000)
    return jax.lax.bitcast_convert_type(fb, jnp.float32) - jnp.float32(1.0)


def _uniform01(k1, k2):
    z = jnp.zeros_like(k1)
    b1, b2 = _tf2x32(k1, k2, z, z)
    f = _bits_to_unit(b1 ^ b2)
    return jnp.maximum(jnp.float32(0.0), f)


_ERFINV_LO = (2.81022636e-08, 3.43273939e-07, -3.5233877e-06, -4.39150654e-06,
              0.00021858087, -0.00125372503, -0.00417768164, 0.246640727,
              1.50140941)
_ERFINV_HI = (-0.000200214257, 0.000100950558, 0.00134934322, -0.00367342844,
              0.00573950773, -0.0076224613, 0.00943887047, 1.00167406,
              2.83297682)


def _erf_inv(x):
    w = -jnp.log1p(-x * x)
    lo_w = w - jnp.float32(2.5)
    hi_w = jnp.sqrt(w) - jnp.float32(3.0)
    p_lo = jnp.full_like(x, np.float32(_ERFINV_LO[0]))
    for cc in _ERFINV_LO[1:]:
        p_lo = np.float32(cc) + p_lo * lo_w
    p_hi = jnp.full_like(x, np.float32(_ERFINV_HI[0]))
    for cc in _ERFINV_HI[1:]:
        p_hi = np.float32(cc) + p_hi * hi_w
    p = jnp.where(w < jnp.float32(5.0), p_lo, p_hi)
    return p * x


_NORM_LO = np.float32(np.nextafter(np.float32(-1.0), np.float32(0.0)))
_NORM_SCALE = np.float32(np.float32(1.0) - _NORM_LO)
_SQRT2 = np.float32(np.sqrt(2.0))


def _normal(k1, k2):
    z = jnp.zeros_like(k1)
    b1, b2 = _tf2x32(k1, k2, z, z)
    f = _bits_to_unit(b1 ^ b2)
    u = jnp.maximum(_NORM_LO, f * _NORM_SCALE + _NORM_LO)
    return _SQRT2 * _erf_inv(u)


_LN2 = np.float32(0.6931471805599453)
_SQRT_HALF = np.float32(np.sqrt(0.5))


def _soft_log(x):
    bits = jax.lax.bitcast_convert_type(x, jnp.uint32)
    e_raw = jax.lax.shift_right_logical(bits, np.uint32(23))
    e_f = e_raw.astype(jnp.float32)
    m_bits = (bits & np.uint32(0x007FFFFF)) | np.uint32(0x3F800000)
    m = jax.lax.bitcast_convert_type(m_bits, jnp.float32)
    small = m > np.float32(1.4142135)
    m = jnp.where(small, m * np.float32(0.5), m)
    e = e_f - np.float32(127.0) + jnp.where(small, jnp.float32(1.0),
                                            jnp.float32(0.0))
    f1 = jnp.float32(1.0)
    s = (m - f1) / (m + f1)
    s2 = s * s
    p = np.float32(1.0 / 9.0)
    p = np.float32(1.0 / 7.0) + p * s2
    p = np.float32(1.0 / 5.0) + p * s2
    p = np.float32(1.0 / 3.0) + p * s2
    p = f1 + p * s2
    r = jnp.float32(2.0) * s * p + e * _LN2
    return jnp.where(x <= 0, jnp.float32(-np.inf), r)


def _soft_log1p(y):
    t = jnp.float32(1.0) + y
    corr = jnp.where(t > 0, (y - (t - jnp.float32(1.0))) / t, jnp.float32(0.0))
    return _soft_log(t) + corr


def _soft_sqrt(d):
    bits = jax.lax.bitcast_convert_type(d, jnp.uint32)
    yb = np.uint32(0x5F3759DF) - jax.lax.shift_right_logical(bits, np.uint32(1))
    y = jax.lax.bitcast_convert_type(yb, jnp.float32)
    half_d = jnp.float32(0.5) * d
    for _ in range(3):
        y = y * (jnp.float32(1.5) - half_d * y * y)
    return d * y


def _erf_inv_soft(x):
    w = -_soft_log1p(-x * x)
    lo_w = w - jnp.float32(2.5)
    hi_w = _soft_sqrt(jnp.maximum(w, jnp.float32(1e-30))) - jnp.float32(3.0)
    p_lo = jnp.full_like(x, np.float32(_ERFINV_LO[0]))
    for cc in _ERFINV_LO[1:]:
        p_lo = np.float32(cc) + p_lo * lo_w
    p_hi = jnp.full_like(x, np.float32(_ERFINV_HI[0]))
    for cc in _ERFINV_HI[1:]:
        p_hi = np.float32(cc) + p_hi * hi_w
    p = jnp.where(w < jnp.float32(5.0), p_lo, p_hi)
    return p * x


def _normal_soft(k1, k2):
    z = jnp.zeros_like(k1)
    b1, b2 = _tf2x32(k1, k2, z, z)
    f = _bits_to_unit(b1 ^ b2)
    u = jnp.maximum(_NORM_LO, f * _NORM_SCALE + _NORM_LO)
    return _SQRT2 * _erf_inv_soft(u)


def _loggamma(gk1, gk2, alpha, log=jnp.log, log1p=jnp.log1p,
              sqrt=jnp.sqrt, normal=_normal):
    z = jnp.zeros_like(gk1)
    one_u = z + np.uint32(1)
    two_u = z + np.uint32(2)
    f1 = jnp.float32(1.0)

    a1, a2 = _tf2x32(gk1, gk2, z, z)
    s1, s2 = _tf2x32(gk1, gk2, z, one_u)

    boost = alpha >= f1
    alpha_b = jnp.where(boost, alpha, alpha + f1)
    d = alpha_b - jnp.float32(1.0 / 3.0)
    c = jnp.float32(1.0 / 3.0) / sqrt(d)

    def reject(X, V, U):
        c1 = U >= f1 - jnp.float32(0.0331) * (X * X)
        c2 = log(U) >= X * jnp.float32(0.5) + d * ((f1 - V) + log(V))
        return c1 & c2

    def draw_v(xk1, xk2):
        sk1, sk2 = _tf2x32(xk1, xk2, z, one_u)
        xn = normal(sk1, sk2)
        return xn, f1 + xn * c

    def inner(xk1, xk2):
        x, v = draw_v(xk1, xk2)

        def inner_cond(ic):
            return jnp.any(ic[3] != 0)

        def inner_body(ic):
            xk1, xk2, x, acti = ic
            act = acti != 0
            nxk1, nxk2 = _tf2x32(xk1, xk2, z, z)
            xk1 = jnp.where(act, nxk1, xk1)
            xk2 = jnp.where(act, nxk2, xk2)
            xn, vn = draw_v(xk1, xk2)
            x = jnp.where(act, xn, x)
            nact = act & (vn <= 0)
            return xk1, xk2, x, jnp.where(nact, np.int32(1), np.int32(0))

        _, _, x, _ = jax.lax.while_loop(
            inner_cond, inner_body,
            (xk1, xk2, x, jnp.where(v <= 0, np.int32(1), np.int32(0))))
        return x

    def one_round(k1, k2):
        xk1, xk2 = _tf2x32(k1, k2, z, one_u)
        uk1, uk2 = _tf2x32(k1, k2, z, two_u)
        x = inner(xk1, xk2)
        Xn = x * x
        Vn = x * c + f1
        Vn = (Vn * Vn) * Vn
        Un = _uniform01(uk1, uk2)
        return Xn, Vn, Un

    X1, V1, U1 = one_round(a1, a2)
    m1 = reject(X1, V1, U1)

    def outer_cond(carry):
        return jnp.any(carry[3] != 0)

    def outer_body(carry):
        k1, k2, V, mi = carry
        m = mi != 0
        nk1, nk2 = _tf2x32(k1, k2, z, z)
        k1 = jnp.where(m, nk1, k1)
        k2 = jnp.where(m, nk2, k2)
        Xn, Vn, Un = one_round(k1, k2)
        V = jnp.where(m, Vn, V)
        nm = m & reject(Xn, Vn, Un)
        return k1, k2, V, jnp.where(nm, np.int32(1), np.int32(0))

    _, _, V, _ = jax.lax.while_loop(
        outer_cond, outer_body,
        (a1, a2, V1, jnp.where(m1, np.int32(1), np.int32(0))))

    u_exp = _uniform01(s1, s2)
    log_samples = log1p(-u_exp)
    log_boost = jnp.where(boost | (log_samples == 0), jnp.float32(0.0),
                          log_samples * (f1 / alpha))
    return (log(d) + log(V)) + log_boost


_LANES = 128
_BLOCK_ROWS = 64
_SUB_ROWS = 8


def _ts_kernel(n_total, block_elems, sub_rows):
    n_sub = _BLOCK_ROWS // sub_rows

    def body(alpha_ref, beta_ref, scores_ref, bmax_ref, bidx_ref):
        g = pl.program_id(0)
        base = (g * np.int32(block_elems)).astype(jnp.int32)
        blk_max = jnp.float32(-2.0)
        blk_idx = jnp.int32(0)
        shape = (sub_rows, _LANES)
        row_i = jax.lax.broadcasted_iota(jnp.int32, shape, 0)
        col_i = jax.lax.broadcasted_iota(jnp.int32, shape, 1)

        for j in range(n_sub):
            a = alpha_ref[j * sub_rows:(j + 1) * sub_rows, :]
            b = beta_ref[j * sub_rows:(j + 1) * sub_rows, :]
            base_j = base + np.int32(j * sub_rows * _LANES)
            lin_i = base_j + row_i * np.int32(_LANES) + col_i
            lin_u = lin_i.astype(jnp.uint32)

            zu = jnp.zeros_like(lin_u)
            ga1, ga2 = _tf2x32(jnp.full(shape, np.uint32(_KA1)),
                               jnp.full(shape, np.uint32(_KA2)), zu, lin_u)
            gb1, gb2 = _tf2x32(jnp.full(shape, np.uint32(_KB1)),
                               jnp.full(shape, np.uint32(_KB2)), zu, lin_u)

            lga = _loggamma(ga1, ga2, a)
            lgb = _loggamma(gb1, gb2, b)
            log_max = jnp.maximum(lga, lgb)
            sa = jnp.exp(lga - log_max)
            sb = jnp.exp(lgb - log_max)
            scores = sa / (sa + sb)
            scores_ref[j * sub_rows:(j + 1) * sub_rows, :] = scores

            valid = lin_i < np.int32(n_total)
            sc = jnp.where(valid, scores, jnp.float32(-1.0))
            m_j = jnp.max(sc)
            i_j = jnp.min(jnp.where(sc == m_j, lin_i, np.int32(2**31 - 1)))
            take = m_j > blk_max
            blk_idx = jnp.where(take, i_j, blk_idx)
            blk_max = jnp.where(take, m_j, blk_max)

        bmax_ref[0, 0, 0] = blk_max
        bidx_ref[0, 0, 0] = blk_idx

    return body


def _argmax_combine(grid_n):
    def body(bmax_ref, bidx_ref, smax_ref, sidx_ref, action_ref):
        def step(g, carry):
            bv, bi = carry
            mv = bmax_ref[g, 0, 0]
            mi = bidx_ref[g, 0, 0]
            take = mv > bv
            return (jnp.where(take, mv, bv), jnp.where(take, mi, bi))

        tc_v, tc_i = jax.lax.fori_loop(
            0, grid_n, step, (jnp.float32(-2.0), jnp.int32(0)))

        sm = smax_ref[...]
        si = sidx_ref[...]
        sc_v = jnp.max(sm)
        sc_i = jnp.min(jnp.where(sm == sc_v, si, np.int32(2**31 - 1)))
        take_sc = sc_v > tc_v
        action_ref[0] = jnp.where(take_sc, sc_i, tc_i)

    return body


_SC_W = 32
_SC_L = 16
_SC_CHUNK = _SC_W * _SC_L


_SC_ROUNDS = 3
_SC_UNROLL = 2


def _loggamma_fixed(gk1, gk2, alpha):
    z = jnp.zeros_like(gk1)
    one_u = z + np.uint32(1)
    two_u = z + np.uint32(2)
    f1 = jnp.float32(1.0)

    a1, a2 = _tf2x32(gk1, gk2, z, z)
    s1, s2 = _tf2x32(gk1, gk2, z, one_u)

    boost = alpha >= f1
    alpha_b = jnp.where(boost, alpha, alpha + f1)
    d = alpha_b - jnp.float32(1.0 / 3.0)
    c = jnp.float32(1.0 / 3.0) / _soft_sqrt(d)

    def reject(X, V, U):
        c1 = U >= f1 - jnp.float32(0.0331) * (X * X)
        c2 = _soft_log(U) >= X * jnp.float32(0.5) + d * ((f1 - V) +
                                                         _soft_log(V))
        return c1 & c2

    def one_round(k1, k2):
        xk1, xk2 = _tf2x32(k1, k2, z, one_u)
        uk1, uk2 = _tf2x32(k1, k2, z, two_u)
        sk1, sk2 = _tf2x32(xk1, xk2, z, one_u)
        x = _normal_soft(sk1, sk2)
        Xn = x * x
        Vn = x * c + f1
        Vn = (Vn * Vn) * Vn
        Un = _uniform01(uk1, uk2)
        return Xn, Vn, Un

    X1, V1, U1 = one_round(a1, a2)
    m1 = reject(X1, V1, U1)

    def round_body(_, carry):
        k1, k2, V, mi = carry
        m = mi != 0
        nk1, nk2 = _tf2x32(k1, k2, z, z)
        k1 = jnp.where(m, nk1, k1)
        k2 = jnp.where(m, nk2, k2)
        Xn, Vn, Un = one_round(k1, k2)
        V = jnp.where(m, Vn, V)
        nm = m & reject(Xn, Vn, Un)
        return k1, k2, V, jnp.where(nm, np.int32(1), np.int32(0))

    _, _, V, _ = jax.lax.fori_loop(
        0, _SC_ROUNDS - 1, round_body,
        (a1, a2, V1, jnp.where(m1, np.int32(1), np.int32(0))))

    u_exp = _uniform01(s1, s2)
    log_samples = _soft_log1p(-u_exp)
    log_boost = jnp.where(boost | (log_samples == 0), jnp.float32(0.0),
                          log_samples * (f1 / alpha))
    return (_soft_log(d) + _soft_log(V)) + log_boost


def _sc_body(n_tc, per_tile):

    def body(alpha_hbm, beta_hbm, scores_hbm, smax_hbm, sidx_hbm,
             a_v, b_v, s_v, stv_v, sti_v, mx_v, mi_v):
        cid = jax.lax.axis_index("c")
        sid = jax.lax.axis_index("s")
        wid = sid * 2 + cid
        base = wid * per_tile
        pltpu.sync_copy(alpha_hbm.at[pl.ds(base, per_tile)], a_v)
        pltpu.sync_copy(beta_hbm.at[pl.ds(base, per_tile)], b_v)

        lane = jax.lax.iota(jnp.int32, _SC_L)
        lane_u = jax.lax.iota(jnp.uint32, _SC_L)
        base_u = base.astype(jnp.uint32)

        def step(g, carry):
            bm, bi = carry
            for j in range(_SC_UNROLL):
                off = (g * _SC_UNROLL + j) * _SC_L
                a = a_v[pl.ds(off, _SC_L)]
                b = b_v[pl.ds(off, _SC_L)]
                lin = (np.int32(n_tc) + base + off) + lane
                lin_u = base_u + np.uint32(n_tc) + jnp.uint32(off) + lane_u
                zu = jnp.zeros_like(lin_u)
                ka1 = zu + np.uint32(_KA1)
                ka2 = zu + np.uint32(_KA2)
                kb1 = zu + np.uint32(_KB1)
                kb2 = zu + np.uint32(_KB2)
                ga1, ga2 = _tf2x32(ka1, ka2, zu, lin_u)
                gb1, gb2 = _tf2x32(kb1, kb2, zu, lin_u)
                lga = _loggamma_fixed(ga1, ga2, a)
                lgb = _loggamma_fixed(gb1, gb2, b)
                log_max = jnp.maximum(lga, lgb)
                sa = jnp.exp(lga - log_max)
                sb = jnp.exp(lgb - log_max)
                sc = sa / (sa + sb)
                s_v[pl.ds(off, _SC_L)] = sc
                upd = sc > bm
                bm = jnp.where(upd, sc, bm)
                bi = jnp.where(upd, lin, bi)
            return bm, bi

        bm0 = jnp.full((_SC_L,), -2.0, jnp.float32)
        bi0 = jnp.zeros((_SC_L,), jnp.int32)
        bm, bi = jax.lax.fori_loop(
            0, per_tile // (_SC_L * _SC_UNROLL), step, (bm0, bi0))

        stv_v[...] = bm
        sti_v[...] = bi
        pltpu.sync_copy(s_v, scores_hbm.at[pl.ds(base, per_tile)])
        pltpu.sync_copy(stv_v, smax_hbm.at[wid])
        pltpu.sync_copy(sti_v, sidx_hbm.at[wid])

    return body


def _sc_sample(alpha_sc, beta_sc, n_tc):
    import functools
    from jax.experimental.pallas import tpu_sc as plsc
    n_sc = alpha_sc.shape[0]
    per_tile = n_sc // _SC_W
    mesh = plsc.VectorSubcoreMesh(core_axis_name="c", subcore_axis_name="s")
    k = functools.partial(
        pl.kernel,
        mesh=mesh,
        out_type=[
            jax.ShapeDtypeStruct((n_sc,), jnp.float32),
            jax.ShapeDtypeStruct((_SC_W, _SC_L), jnp.float32),
            jax.ShapeDtypeStruct((_SC_W, _SC_L), jnp.int32),
        ],
        scratch_types=[
            pltpu.VMEM((per_tile,), jnp.float32),
            pltpu.VMEM((per_tile,), jnp.float32),
            pltpu.VMEM((per_tile,), jnp.float32),
            pltpu.VMEM((_SC_L,), jnp.float32),
            pltpu.VMEM((_SC_L,), jnp.int32),
            pltpu.VMEM((_SC_L,), jnp.float32),
            pltpu.VMEM((_SC_L,), jnp.int32),
        ],
    )(_sc_body(n_tc, per_tile))
    return k(alpha_sc, beta_sc)


_SC_FRAC_NUM = 1
_SC_FRAC_DEN = 8


def kernel(alpha, beta):
    n = alpha.shape[0]
    n_sc = ((n * _SC_FRAC_NUM) // _SC_FRAC_DEN) // _SC_CHUNK * _SC_CHUNK
    n_tc = n - n_sc
    return _kernel_split(alpha, beta, n, n_tc, n_sc)


def _kernel_split(alpha, beta, n, n_tc, n_sc):
    rows = -(-n_tc // _LANES)
    rows_pad = -(-rows // _BLOCK_ROWS) * _BLOCK_ROWS
    total = rows_pad * _LANES
    grid_n = rows_pad // _BLOCK_ROWS
    block_elems = _BLOCK_ROWS * _LANES

    a2d = jnp.concatenate(
        [alpha[:n_tc],
         jnp.ones((total - n_tc,), jnp.float32)]).reshape(rows_pad, _LANES)
    b2d = jnp.concatenate(
        [beta[:n_tc],
         jnp.ones((total - n_tc,), jnp.float32)]).reshape(rows_pad, _LANES)

    scores2d, bmax, bidx = pl.pallas_call(
        _ts_kernel(n_tc, block_elems, _SUB_ROWS),
        grid=(grid_n,),
        in_specs=[
            pl.BlockSpec((_BLOCK_ROWS, _LANES), lambda g: (g, 0)),
            pl.BlockSpec((_BLOCK_ROWS, _LANES), lambda g: (g, 0)),
        ],
        out_specs=[
            pl.BlockSpec((_BLOCK_ROWS, _LANES), lambda g: (g, 0)),
            pl.BlockSpec((1, 1, 1), lambda g: (g, 0, 0), memory_space=pltpu.SMEM),
            pl.BlockSpec((1, 1, 1), lambda g: (g, 0, 0), memory_space=pltpu.SMEM),
        ],
        out_shape=[
            jax.ShapeDtypeStruct((rows_pad, _LANES), jnp.float32),
            jax.ShapeDtypeStruct((grid_n, 1, 1), jnp.float32),
            jax.ShapeDtypeStruct((grid_n, 1, 1), jnp.int32),
        ],
        compiler_params=pltpu.CompilerParams(
            dimension_semantics=("parallel",)),
    )(a2d, b2d)

    if n_sc:
        sc_scores, smax, sidx = _sc_sample(alpha[n_tc:], beta[n_tc:], n_tc)
    else:
        sc_scores = jnp.zeros((0,), jnp.float32)
        smax = jnp.full((_SC_W, _SC_L), -2.0, jnp.float32)
        sidx = jnp.zeros((_SC_W, _SC_L), jnp.int32)

    action1 = pl.pallas_call(
        _argmax_combine(grid_n),
        in_specs=[
            pl.BlockSpec(memory_space=pltpu.SMEM),
            pl.BlockSpec(memory_space=pltpu.SMEM),
            pl.BlockSpec((4, _SC_W * _SC_L // 4), lambda: (0, 0)),
            pl.BlockSpec((4, _SC_W * _SC_L // 4), lambda: (0, 0)),
        ],
        out_specs=pl.BlockSpec(memory_space=pltpu.SMEM),
        out_shape=jax.ShapeDtypeStruct((1,), jnp.int32),
    )(bmax, bidx,
      smax.reshape(4, _SC_W * _SC_L // 4),
      sidx.reshape(4, _SC_W * _SC_L // 4))

    scores = jnp.concatenate([scores2d.reshape(-1)[:n_tc], sc_scores])
    action = action1[0]
    return (action, scores)


# speedup vs baseline: 2.8926x; 2.8926x over previous
import numpy as np
import jax
import jax.numpy as jnp
from jax.experimental import pallas as pl
from jax.experimental.pallas import tpu as pltpu

_MAGIC = 0x1BD11BDA
_R1 = (13, 15, 26, 6)
_R2 = (17, 29, 16, 24)
_M32 = 0xFFFFFFFF


def _tf2x32_py(k1, k2, c1, c2):
    ks0, ks1 = k1, k2
    ks2 = k1 ^ k2 ^ _MAGIC
    x0 = (c1 + ks0) & _M32
    x1 = (c2 + ks1) & _M32

    def four(x0, x1, rs):
        for r in rs:
            x0 = (x0 + x1) & _M32
            x1 = ((x1 << r) | (x1 >> (32 - r))) & _M32
            x1 = x0 ^ x1
        return x0, x1

    x0, x1 = four(x0, x1, _R1); x0 = (x0 + ks1) & _M32; x1 = (x1 + ks2 + 1) & _M32
    x0, x1 = four(x0, x1, _R2); x0 = (x0 + ks2) & _M32; x1 = (x1 + ks0 + 2) & _M32
    x0, x1 = four(x0, x1, _R1); x0 = (x0 + ks0) & _M32; x1 = (x1 + ks1 + 3) & _M32
    x0, x1 = four(x0, x1, _R2); x0 = (x0 + ks1) & _M32; x1 = (x1 + ks2 + 4) & _M32
    x0, x1 = four(x0, x1, _R1); x0 = (x0 + ks2) & _M32; x1 = (x1 + ks0 + 5) & _M32
    return x0, x1


_KA1, _KA2 = _tf2x32_py(0, 42, 0, 0)
_KB1, _KB2 = _tf2x32_py(0, 42, 0, 1)


def _tf2x32(k1, k2, c1, c2):
    sl = jax.lax.shift_left
    sr = jax.lax.shift_right_logical
    ks0, ks1 = k1, k2
    ks2 = k1 ^ k2 ^ np.uint32(_MAGIC)
    x0 = c1 + ks0
    x1 = c2 + ks1

    def four(x0, x1, rs):
        for r in rs:
            x0 = x0 + x1
            x1 = sl(x1, np.uint32(r)) | sr(x1, np.uint32(32 - r))
            x1 = x0 ^ x1
        return x0, x1

    x0, x1 = four(x0, x1, _R1); x0 = x0 + ks1; x1 = x1 + (ks2 + np.uint32(1))
    x0, x1 = four(x0, x1, _R2); x0 = x0 + ks2; x1 = x1 + (ks0 + np.uint32(2))
    x0, x1 = four(x0, x1, _R1); x0 = x0 + ks0; x1 = x1 + (ks1 + np.uint32(3))
    x0, x1 = four(x0, x1, _R2); x0 = x0 + ks1; x1 = x1 + (ks2 + np.uint32(4))
    x0, x1 = four(x0, x1, _R1); x0 = x0 + ks2; x1 = x1 + (ks0 + np.uint32(5))
    return x0, x1


def _bits_to_unit(bits):
    fb = jax.lax.shift_right_logical(bits, np.uint32(9)) | np.uint32(0x3F800000)
    return jax.lax.bitcast_convert_type(fb, jnp.float32) - jnp.float32(1.0)


def _uniform01(k1, k2):
    z = jnp.zeros_like(k1)
    b1, b2 = _tf2x32(k1, k2, z, z)
    f = _bits_to_unit(b1 ^ b2)
    return jnp.maximum(jnp.float32(0.0), f)


_ERFINV_LO = (2.81022636e-08, 3.43273939e-07, -3.5233877e-06, -4.39150654e-06,
              0.00021858087, -0.00125372503, -0.00417768164, 0.246640727,
              1.50140941)
_ERFINV_HI = (-0.000200214257, 0.000100950558, 0.00134934322, -0.00367342844,
              0.00573950773, -0.0076224613, 0.00943887047, 1.00167406,
              2.83297682)


def _erf_inv(x):
    w = -jnp.log1p(-x * x)
    lo_w = w - jnp.float32(2.5)
    hi_w = jnp.sqrt(w) - jnp.float32(3.0)
    p_lo = jnp.full_like(x, np.float32(_ERFINV_LO[0]))
    for cc in _ERFINV_LO[1:]:
        p_lo = np.float32(cc) + p_lo * lo_w
    p_hi = jnp.full_like(x, np.float32(_ERFINV_HI[0]))
    for cc in _ERFINV_HI[1:]:
        p_hi = np.float32(cc) + p_hi * hi_w
    p = jnp.where(w < jnp.float32(5.0), p_lo, p_hi)
    return p * x


_NORM_LO = np.float32(np.nextafter(np.float32(-1.0), np.float32(0.0)))
_NORM_SCALE = np.float32(np.float32(1.0) - _NORM_LO)
_SQRT2 = np.float32(np.sqrt(2.0))


def _normal(k1, k2):
    z = jnp.zeros_like(k1)
    b1, b2 = _tf2x32(k1, k2, z, z)
    f = _bits_to_unit(b1 ^ b2)
    u = jnp.maximum(_NORM_LO, f * _NORM_SCALE + _NORM_LO)
    return _SQRT2 * _erf_inv(u)


_LN2 = np.float32(0.6931471805599453)
_SQRT_HALF = np.float32(np.sqrt(0.5))


def _soft_log(x):
    bits = jax.lax.bitcast_convert_type(x, jnp.uint32)
    e_raw = jax.lax.shift_right_logical(bits, np.uint32(23))
    e_f = e_raw.astype(jnp.float32)
    m_bits = (bits & np.uint32(0x007FFFFF)) | np.uint32(0x3F800000)
    m = jax.lax.bitcast_convert_type(m_bits, jnp.float32)
    small = m > np.float32(1.4142135)
    m = jnp.where(small, m * np.float32(0.5), m)
    e = e_f - np.float32(127.0) + jnp.where(small, jnp.float32(1.0),
                                            jnp.float32(0.0))
    f1 = jnp.float32(1.0)
    s = (m - f1) / (m + f1)
    s2 = s * s
    p = np.float32(1.0 / 9.0)
    p = np.float32(1.0 / 7.0) + p * s2
    p = np.float32(1.0 / 5.0) + p * s2
    p = np.float32(1.0 / 3.0) + p * s2
    p = f1 + p * s2
    r = jnp.float32(2.0) * s * p + e * _LN2
    return jnp.where(x <= 0, jnp.float32(-np.inf), r)


def _soft_log1p(y):
    t = jnp.float32(1.0) + y
    corr = jnp.where(t > 0, (y - (t - jnp.float32(1.0))) / t, jnp.float32(0.0))
    return _soft_log(t) + corr


def _soft_sqrt(d):
    bits = jax.lax.bitcast_convert_type(d, jnp.uint32)
    yb = np.uint32(0x5F3759DF) - jax.lax.shift_right_logical(bits, np.uint32(1))
    y = jax.lax.bitcast_convert_type(yb, jnp.float32)
    half_d = jnp.float32(0.5) * d
    for _ in range(3):
        y = y * (jnp.float32(1.5) - half_d * y * y)
    return d * y


def _erf_inv_soft(x):
    w = -_soft_log1p(-x * x)
    lo_w = w - jnp.float32(2.5)
    hi_w = _soft_sqrt(jnp.maximum(w, jnp.float32(1e-30))) - jnp.float32(3.0)
    p_lo = jnp.full_like(x, np.float32(_ERFINV_LO[0]))
    for cc in _ERFINV_LO[1:]:
        p_lo = np.float32(cc) + p_lo * lo_w
    p_hi = jnp.full_like(x, np.float32(_ERFINV_HI[0]))
    for cc in _ERFINV_HI[1:]:
        p_hi = np.float32(cc) + p_hi * hi_w
    p = jnp.where(w < jnp.float32(5.0), p_lo, p_hi)
    return p * x


def _normal_soft(k1, k2):
    z = jnp.zeros_like(k1)
    b1, b2 = _tf2x32(k1, k2, z, z)
    f = _bits_to_unit(b1 ^ b2)
    u = jnp.maximum(_NORM_LO, f * _NORM_SCALE + _NORM_LO)
    return _SQRT2 * _erf_inv_soft(u)


def _loggamma(gk1, gk2, alpha, log=jnp.log, log1p=jnp.log1p,
              sqrt=jnp.sqrt, normal=_normal):
    z = jnp.zeros_like(gk1)
    one_u = z + np.uint32(1)
    two_u = z + np.uint32(2)
    f1 = jnp.float32(1.0)

    a1, a2 = _tf2x32(gk1, gk2, z, z)
    s1, s2 = _tf2x32(gk1, gk2, z, one_u)

    boost = alpha >= f1
    alpha_b = jnp.where(boost, alpha, alpha + f1)
    d = alpha_b - jnp.float32(1.0 / 3.0)
    c = jnp.float32(1.0 / 3.0) / sqrt(d)

    def reject(X, V, U):
        c1 = U >= f1 - jnp.float32(0.0331) * (X * X)
        c2 = log(U) >= X * jnp.float32(0.5) + d * ((f1 - V) + log(V))
        return c1 & c2

    def draw_v(xk1, xk2):
        sk1, sk2 = _tf2x32(xk1, xk2, z, one_u)
        xn = normal(sk1, sk2)
        return xn, f1 + xn * c

    def inner(xk1, xk2):
        x, v = draw_v(xk1, xk2)

        def inner_cond(ic):
            return jnp.any(ic[3] != 0)

        def inner_body(ic):
            xk1, xk2, x, acti = ic
            act = acti != 0
            nxk1, nxk2 = _tf2x32(xk1, xk2, z, z)
            xk1 = jnp.where(act, nxk1, xk1)
            xk2 = jnp.where(act, nxk2, xk2)
            xn, vn = draw_v(xk1, xk2)
            x = jnp.where(act, xn, x)
            nact = act & (vn <= 0)
            return xk1, xk2, x, jnp.where(nact, np.int32(1), np.int32(0))

        _, _, x, _ = jax.lax.while_loop(
            inner_cond, inner_body,
            (xk1, xk2, x, jnp.where(v <= 0, np.int32(1), np.int32(0))))
        return x

    def one_round(k1, k2):
        xk1, xk2 = _tf2x32(k1, k2, z, one_u)
        uk1, uk2 = _tf2x32(k1, k2, z, two_u)
        x = inner(xk1, xk2)
        Xn = x * x
        Vn = x * c + f1
        Vn = (Vn * Vn) * Vn
        Un = _uniform01(uk1, uk2)
        return Xn, Vn, Un

    X1, V1, U1 = one_round(a1, a2)
    m1 = reject(X1, V1, U1)

    def outer_cond(carry):
        return jnp.any(carry[3] != 0)

    def outer_body(carry):
        k1, k2, V, mi = carry
        m = mi != 0
        nk1, nk2 = _tf2x32(k1, k2, z, z)
        k1 = jnp.where(m, nk1, k1)
        k2 = jnp.where(m, nk2, k2)
        Xn, Vn, Un = one_round(k1, k2)
        V = jnp.where(m, Vn, V)
        nm = m & reject(Xn, Vn, Un)
        return k1, k2, V, jnp.where(nm, np.int32(1), np.int32(0))

    _, _, V, _ = jax.lax.while_loop(
        outer_cond, outer_body,
        (a1, a2, V1, jnp.where(m1, np.int32(1), np.int32(0))))

    u_exp = _uniform01(s1, s2)
    log_samples = log1p(-u_exp)
    log_boost = jnp.where(boost | (log_samples == 0), jnp.float32(0.0),
                          log_samples * (f1 / alpha))
    return (log(d) + log(V)) + log_boost


_LANES = 128
_BLOCK_ROWS = 64
_SUB_ROWS = 8


def _ts_kernel(n_total, block_elems):
    def body(alpha_ref, beta_ref, scores_ref, bmax_ref, bidx_ref):
        g = pl.program_id(0)
        a = alpha_ref[...]
        b = beta_ref[...]
        shape = a.shape

        base = (g * np.int32(block_elems)).astype(jnp.int32)
        row_i = jax.lax.broadcasted_iota(jnp.int32, shape, 0)
        col_i = jax.lax.broadcasted_iota(jnp.int32, shape, 1)
        lin_i = base + row_i * np.int32(_LANES) + col_i
        lin_u = lin_i.astype(jnp.uint32)

        zu = jnp.zeros_like(lin_u)
        ga1, ga2 = _tf2x32(jnp.full(shape, np.uint32(_KA1)),
                           jnp.full(shape, np.uint32(_KA2)), zu, lin_u)
        gb1, gb2 = _tf2x32(jnp.full(shape, np.uint32(_KB1)),
                           jnp.full(shape, np.uint32(_KB2)), zu, lin_u)

        lga = _loggamma(ga1, ga2, a)
        lgb = _loggamma(gb1, gb2, b)
        log_max = jnp.maximum(lga, lgb)
        sa = jnp.exp(lga - log_max)
        sb = jnp.exp(lgb - log_max)
        scores = sa / (sa + sb)
        scores_ref[...] = scores

        valid = lin_i < np.int32(n_total)
        sc = jnp.where(valid, scores, jnp.float32(-1.0))
        blk_max = jnp.max(sc)
        blk_idx = jnp.min(jnp.where(sc == blk_max, lin_i, np.int32(2**31 - 1)))
        bmax_ref[0, 0, 0] = blk_max
        bidx_ref[0, 0, 0] = blk_idx

    return body


def _argmax_combine(grid_n):
    def body(bmax_ref, bidx_ref, smax_ref, sidx_ref, action_ref):
        def step(g, carry):
            bv, bi = carry
            mv = bmax_ref[g, 0, 0]
            mi = bidx_ref[g, 0, 0]
            take = mv > bv
            return (jnp.where(take, mv, bv), jnp.where(take, mi, bi))

        tc_v, tc_i = jax.lax.fori_loop(
            0, grid_n, step, (jnp.float32(-2.0), jnp.int32(0)))

        sm = smax_ref[...]
        si = sidx_ref[...]
        sc_v = jnp.max(sm)
        sc_i = jnp.min(jnp.where(sm == sc_v, si, np.int32(2**31 - 1)))
        take_sc = sc_v > tc_v
        action_ref[0] = jnp.where(take_sc, sc_i, tc_i)

    return body


_SC_W = 32
_SC_L = 16
_SC_CHUNK = _SC_W * _SC_L


_SC_ROUNDS = 3
_SC_UNROLL = 2


def _loggamma_fixed(gk1, gk2, alpha):
    z = jnp.zeros_like(gk1)
    one_u = z + np.uint32(1)
    two_u = z + np.uint32(2)
    f1 = jnp.float32(1.0)

    a1, a2 = _tf2x32(gk1, gk2, z, z)
    s1, s2 = _tf2x32(gk1, gk2, z, one_u)

    boost = alpha >= f1
    alpha_b = jnp.where(boost, alpha, alpha + f1)
    d = alpha_b - jnp.float32(1.0 / 3.0)
    c = jnp.float32(1.0 / 3.0) / _soft_sqrt(d)

    def reject(X, V, U):
        c1 = U >= f1 - jnp.float32(0.0331) * (X * X)
        c2 = _soft_log(U) >= X * jnp.float32(0.5) + d * ((f1 - V) +
                                                         _soft_log(V))
        return c1 & c2

    def one_round(k1, k2):
        xk1, xk2 = _tf2x32(k1, k2, z, one_u)
        uk1, uk2 = _tf2x32(k1, k2, z, two_u)
        sk1, sk2 = _tf2x32(xk1, xk2, z, one_u)
        x = _normal_soft(sk1, sk2)
        Xn = x * x
        Vn = x * c + f1
        Vn = (Vn * Vn) * Vn
        Un = _uniform01(uk1, uk2)
        return Xn, Vn, Un

    X1, V1, U1 = one_round(a1, a2)
    m1 = reject(X1, V1, U1)

    def round_body(_, carry):
        k1, k2, V, mi = carry
        m = mi != 0
        nk1, nk2 = _tf2x32(k1, k2, z, z)
        k1 = jnp.where(m, nk1, k1)
        k2 = jnp.where(m, nk2, k2)
        Xn, Vn, Un = one_round(k1, k2)
        V = jnp.where(m, Vn, V)
        nm = m & reject(Xn, Vn, Un)
        return k1, k2, V, jnp.where(nm, np.int32(1), np.int32(0))

    _, _, V, _ = jax.lax.fori_loop(
        0, _SC_ROUNDS - 1, round_body,
        (a1, a2, V1, jnp.where(m1, np.int32(1), np.int32(0))))

    u_exp = _uniform01(s1, s2)
    log_samples = _soft_log1p(-u_exp)
    log_boost = jnp.where(boost | (log_samples == 0), jnp.float32(0.0),
                          log_samples * (f1 / alpha))
    return (_soft_log(d) + _soft_log(V)) + log_boost


def _sc_body(n_tc, per_tile):

    def body(alpha_hbm, beta_hbm, scores_hbm, smax_hbm, sidx_hbm,
             a_v, b_v, s_v, stv_v, sti_v, mx_v, mi_v):
        cid = jax.lax.axis_index("c")
        sid = jax.lax.axis_index("s")
        wid = sid * 2 + cid
        base = wid * per_tile
        pltpu.sync_copy(alpha_hbm.at[pl.ds(base, per_tile)], a_v)
        pltpu.sync_copy(beta_hbm.at[pl.ds(base, per_tile)], b_v)

        lane = jax.lax.iota(jnp.int32, _SC_L)
        lane_u = jax.lax.iota(jnp.uint32, _SC_L)
        base_u = base.astype(jnp.uint32)

        def step(g, carry):
            bm, bi = carry
            for j in range(_SC_UNROLL):
                off = (g * _SC_UNROLL + j) * _SC_L
                a = a_v[pl.ds(off, _SC_L)]
                b = b_v[pl.ds(off, _SC_L)]
                lin = (np.int32(n_tc) + base + off) + lane
                lin_u = base_u + np.uint32(n_tc) + jnp.uint32(off) + lane_u
                zu = jnp.zeros_like(lin_u)
                ka1 = zu + np.uint32(_KA1)
                ka2 = zu + np.uint32(_KA2)
                kb1 = zu + np.uint32(_KB1)
                kb2 = zu + np.uint32(_KB2)
                ga1, ga2 = _tf2x32(ka1, ka2, zu, lin_u)
                gb1, gb2 = _tf2x32(kb1, kb2, zu, lin_u)
                lga = _loggamma_fixed(ga1, ga2, a)
                lgb = _loggamma_fixed(gb1, gb2, b)
                log_max = jnp.maximum(lga, lgb)
                sa = jnp.exp(lga - log_max)
                sb = jnp.exp(lgb - log_max)
                sc = sa / (sa + sb)
                s_v[pl.ds(off, _SC_L)] = sc
                upd = sc > bm
                bm = jnp.where(upd, sc, bm)
                bi = jnp.where(upd, lin, bi)
            return bm, bi

        bm0 = jnp.full((_SC_L,), -2.0, jnp.float32)
        bi0 = jnp.zeros((_SC_L,), jnp.int32)
        bm, bi = jax.lax.fori_loop(
            0, per_tile // (_SC_L * _SC_UNROLL), step, (bm0, bi0))

        stv_v[...] = bm
        sti_v[...] = bi
        pltpu.sync_copy(s_v, scores_hbm.at[pl.ds(base, per_tile)])
        pltpu.sync_copy(stv_v, smax_hbm.at[wid])
        pltpu.sync_copy(sti_v, sidx_hbm.at[wid])

    return body


def _sc_sample(alpha_sc, beta_sc, n_tc):
    import functools
    from jax.experimental.pallas import tpu_sc as plsc
    n_sc = alpha_sc.shape[0]
    per_tile = n_sc // _SC_W
    mesh = plsc.VectorSubcoreMesh(core_axis_name="c", subcore_axis_name="s")
    k = functools.partial(
        pl.kernel,
        mesh=mesh,
        out_type=[
            jax.ShapeDtypeStruct((n_sc,), jnp.float32),
            jax.ShapeDtypeStruct((_SC_W, _SC_L), jnp.float32),
            jax.ShapeDtypeStruct((_SC_W, _SC_L), jnp.int32),
        ],
        scratch_types=[
            pltpu.VMEM((per_tile,), jnp.float32),
            pltpu.VMEM((per_tile,), jnp.float32),
            pltpu.VMEM((per_tile,), jnp.float32),
            pltpu.VMEM((_SC_L,), jnp.float32),
            pltpu.VMEM((_SC_L,), jnp.int32),
            pltpu.VMEM((_SC_L,), jnp.float32),
            pltpu.VMEM((_SC_L,), jnp.int32),
        ],
    )(_sc_body(n_tc, per_tile))
    return k(alpha_sc, beta_sc)


_SC_FRAC_NUM = 1
_SC_FRAC_DEN = 16


def kernel(alpha, beta):
    n = alpha.shape[0]
    n_sc = ((n * _SC_FRAC_NUM) // _SC_FRAC_DEN) // _SC_CHUNK * _SC_CHUNK
    n_tc = n - n_sc
    return _kernel_split(alpha, beta, n, n_tc, n_sc)


def _kernel_split(alpha, beta, n, n_tc, n_sc):
    rows = -(-n_tc // _LANES)
    rows_pad = -(-rows // _BLOCK_ROWS) * _BLOCK_ROWS
    total = rows_pad * _LANES
    grid_n = rows_pad // _BLOCK_ROWS
    block_elems = _BLOCK_ROWS * _LANES

    a2d = jnp.concatenate(
        [alpha[:n_tc],
         jnp.ones((total - n_tc,), jnp.float32)]).reshape(rows_pad, _LANES)
    b2d = jnp.concatenate(
        [beta[:n_tc],
         jnp.ones((total - n_tc,), jnp.float32)]).reshape(rows_pad, _LANES)

    scores2d, bmax, bidx = pl.pallas_call(
        _ts_kernel(n_tc, block_elems),
        grid=(grid_n,),
        in_specs=[
            pl.BlockSpec((_BLOCK_ROWS, _LANES), lambda g: (g, 0)),
            pl.BlockSpec((_BLOCK_ROWS, _LANES), lambda g: (g, 0)),
        ],
        out_specs=[
            pl.BlockSpec((_BLOCK_ROWS, _LANES), lambda g: (g, 0)),
            pl.BlockSpec((1, 1, 1), lambda g: (g, 0, 0), memory_space=pltpu.SMEM),
            pl.BlockSpec((1, 1, 1), lambda g: (g, 0, 0), memory_space=pltpu.SMEM),
        ],
        out_shape=[
            jax.ShapeDtypeStruct((rows_pad, _LANES), jnp.float32),
            jax.ShapeDtypeStruct((grid_n, 1, 1), jnp.float32),
            jax.ShapeDtypeStruct((grid_n, 1, 1), jnp.int32),
        ],
        compiler_params=pltpu.CompilerParams(
            dimension_semantics=("parallel",)),
    )(a2d, b2d)

    if n_sc:
        sc_scores, smax, sidx = _sc_sample(alpha[n_tc:], beta[n_tc:], n_tc)
    else:
        sc_scores = jnp.zeros((0,), jnp.float32)
        smax = jnp.full((_SC_W, _SC_L), -2.0, jnp.float32)
        sidx = jnp.zeros((_SC_W, _SC_L), jnp.int32)

    action1 = pl.pallas_call(
        _argmax_combine(grid_n),
        in_specs=[
            pl.BlockSpec(memory_space=pltpu.SMEM),
            pl.BlockSpec(memory_space=pltpu.SMEM),
            pl.BlockSpec((4, _SC_W * _SC_L // 4), lambda: (0, 0)),
            pl.BlockSpec((4, _SC_W * _SC_L // 4), lambda: (0, 0)),
        ],
        out_specs=pl.BlockSpec(memory_space=pltpu.SMEM),
        out_shape=jax.ShapeDtypeStruct((1,), jnp.int32),
    )(bmax, bidx,
      smax.reshape(4, _SC_W * _SC_L // 4),
      sidx.reshape(4, _SC_W * _SC_L // 4))

    scores = jnp.concatenate([scores2d.reshape(-1)[:n_tc], sc_scores])
    action = action1[0]
    return (action, scores)


# speedup vs baseline: 2.9790x; 1.0299x over previous
import numpy as np
import jax
import jax.numpy as jnp
from jax.experimental import pallas as pl
from jax.experimental.pallas import tpu as pltpu

_MAGIC = 0x1BD11BDA
_R1 = (13, 15, 26, 6)
_R2 = (17, 29, 16, 24)
_M32 = 0xFFFFFFFF


def _tf2x32_py(k1, k2, c1, c2):
    ks0, ks1 = k1, k2
    ks2 = k1 ^ k2 ^ _MAGIC
    x0 = (c1 + ks0) & _M32
    x1 = (c2 + ks1) & _M32

    def four(x0, x1, rs):
        for r in rs:
            x0 = (x0 + x1) & _M32
            x1 = ((x1 << r) | (x1 >> (32 - r))) & _M32
            x1 = x0 ^ x1
        return x0, x1

    x0, x1 = four(x0, x1, _R1); x0 = (x0 + ks1) & _M32; x1 = (x1 + ks2 + 1) & _M32
    x0, x1 = four(x0, x1, _R2); x0 = (x0 + ks2) & _M32; x1 = (x1 + ks0 + 2) & _M32
    x0, x1 = four(x0, x1, _R1); x0 = (x0 + ks0) & _M32; x1 = (x1 + ks1 + 3) & _M32
    x0, x1 = four(x0, x1, _R2); x0 = (x0 + ks1) & _M32; x1 = (x1 + ks2 + 4) & _M32
    x0, x1 = four(x0, x1, _R1); x0 = (x0 + ks2) & _M32; x1 = (x1 + ks0 + 5) & _M32
    return x0, x1


_KA1, _KA2 = _tf2x32_py(0, 42, 0, 0)
_KB1, _KB2 = _tf2x32_py(0, 42, 0, 1)


def _tf2x32(k1, k2, c1, c2):
    sl = jax.lax.shift_left
    sr = jax.lax.shift_right_logical
    ks0, ks1 = k1, k2
    ks2 = k1 ^ k2 ^ np.uint32(_MAGIC)
    x0 = c1 + ks0
    x1 = c2 + ks1

    def four(x0, x1, rs):
        for r in rs:
            x0 = x0 + x1
            x1 = sl(x1, np.uint32(r)) | sr(x1, np.uint32(32 - r))
            x1 = x0 ^ x1
        return x0, x1

    x0, x1 = four(x0, x1, _R1); x0 = x0 + ks1; x1 = x1 + (ks2 + np.uint32(1))
    x0, x1 = four(x0, x1, _R2); x0 = x0 + ks2; x1 = x1 + (ks0 + np.uint32(2))
    x0, x1 = four(x0, x1, _R1); x0 = x0 + ks0; x1 = x1 + (ks1 + np.uint32(3))
    x0, x1 = four(x0, x1, _R2); x0 = x0 + ks1; x1 = x1 + (ks2 + np.uint32(4))
    x0, x1 = four(x0, x1, _R1); x0 = x0 + ks2; x1 = x1 + (ks0 + np.uint32(5))
    return x0, x1


def _bits_to_unit(bits):
    fb = jax.lax.shift_right_logical(bits, np.uint32(9)) | np.uint32(0x3F800000)
    return jax.lax.bitcast_convert_type(fb, jnp.float32) - jnp.float32(1.0)


def _uniform01(k1, k2):
    z = jnp.zeros_like(k1)
    b1, b2 = _tf2x32(k1, k2, z, z)
    f = _bits_to_unit(b1 ^ b2)
    return jnp.maximum(jnp.float32(0.0), f)


_ERFINV_LO = (2.81022636e-08, 3.43273939e-07, -3.5233877e-06, -4.39150654e-06,
              0.00021858087, -0.00125372503, -0.00417768164, 0.246640727,
              1.50140941)
_ERFINV_HI = (-0.000200214257, 0.000100950558, 0.00134934322, -0.00367342844,
              0.00573950773, -0.0076224613, 0.00943887047, 1.00167406,
              2.83297682)


def _erf_inv(x):
    w = -jnp.log1p(-x * x)
    lo_w = w - jnp.float32(2.5)
    hi_w = jnp.sqrt(w) - jnp.float32(3.0)
    p_lo = jnp.full_like(x, np.float32(_ERFINV_LO[0]))
    for cc in _ERFINV_LO[1:]:
        p_lo = np.float32(cc) + p_lo * lo_w
    p_hi = jnp.full_like(x, np.float32(_ERFINV_HI[0]))
    for cc in _ERFINV_HI[1:]:
        p_hi = np.float32(cc) + p_hi * hi_w
    p = jnp.where(w < jnp.float32(5.0), p_lo, p_hi)
    return p * x


_NORM_LO = np.float32(np.nextafter(np.float32(-1.0), np.float32(0.0)))
_NORM_SCALE = np.float32(np.float32(1.0) - _NORM_LO)
_SQRT2 = np.float32(np.sqrt(2.0))


def _normal(k1, k2):
    z = jnp.zeros_like(k1)
    b1, b2 = _tf2x32(k1, k2, z, z)
    f = _bits_to_unit(b1 ^ b2)
    u = jnp.maximum(_NORM_LO, f * _NORM_SCALE + _NORM_LO)
    return _SQRT2 * _erf_inv(u)


_LN2 = np.float32(0.6931471805599453)
_SQRT_HALF = np.float32(np.sqrt(0.5))


def _soft_log(x):
    bits = jax.lax.bitcast_convert_type(x, jnp.uint32)
    e_raw = jax.lax.shift_right_logical(bits, np.uint32(23))
    e_f = e_raw.astype(jnp.float32)
    m_bits = (bits & np.uint32(0x007FFFFF)) | np.uint32(0x3F800000)
    m = jax.lax.bitcast_convert_type(m_bits, jnp.float32)
    small = m > np.float32(1.4142135)
    m = jnp.where(small, m * np.float32(0.5), m)
    e = e_f - np.float32(127.0) + jnp.where(small, jnp.float32(1.0),
                                            jnp.float32(0.0))
    f1 = jnp.float32(1.0)
    s = (m - f1) / (m + f1)
    s2 = s * s
    p = np.float32(1.0 / 9.0)
    p = np.float32(1.0 / 7.0) + p * s2
    p = np.float32(1.0 / 5.0) + p * s2
    p = np.float32(1.0 / 3.0) + p * s2
    p = f1 + p * s2
    r = jnp.float32(2.0) * s * p + e * _LN2
    return jnp.where(x <= 0, jnp.float32(-np.inf), r)


def _soft_log1p(y):
    t = jnp.float32(1.0) + y
    corr = jnp.where(t > 0, (y - (t - jnp.float32(1.0))) / t, jnp.float32(0.0))
    return _soft_log(t) + corr


def _soft_sqrt(d):
    bits = jax.lax.bitcast_convert_type(d, jnp.uint32)
    yb = np.uint32(0x5F3759DF) - jax.lax.shift_right_logical(bits, np.uint32(1))
    y = jax.lax.bitcast_convert_type(yb, jnp.float32)
    half_d = jnp.float32(0.5) * d
    for _ in range(3):
        y = y * (jnp.float32(1.5) - half_d * y * y)
    return d * y


def _erf_inv_soft(x):
    w = -_soft_log1p(-x * x)
    lo_w = w - jnp.float32(2.5)
    hi_w = _soft_sqrt(jnp.maximum(w, jnp.float32(1e-30))) - jnp.float32(3.0)
    p_lo = jnp.full_like(x, np.float32(_ERFINV_LO[0]))
    for cc in _ERFINV_LO[1:]:
        p_lo = np.float32(cc) + p_lo * lo_w
    p_hi = jnp.full_like(x, np.float32(_ERFINV_HI[0]))
    for cc in _ERFINV_HI[1:]:
        p_hi = np.float32(cc) + p_hi * hi_w
    p = jnp.where(w < jnp.float32(5.0), p_lo, p_hi)
    return p * x


def _normal_soft(k1, k2):
    z = jnp.zeros_like(k1)
    b1, b2 = _tf2x32(k1, k2, z, z)
    f = _bits_to_unit(b1 ^ b2)
    u = jnp.maximum(_NORM_LO, f * _NORM_SCALE + _NORM_LO)
    return _SQRT2 * _erf_inv_soft(u)


def _loggamma(gk1, gk2, alpha, log=jnp.log, log1p=jnp.log1p,
              sqrt=jnp.sqrt, normal=_normal):
    z = jnp.zeros_like(gk1)
    one_u = z + np.uint32(1)
    two_u = z + np.uint32(2)
    f1 = jnp.float32(1.0)

    a1, a2 = _tf2x32(gk1, gk2, z, z)
    s1, s2 = _tf2x32(gk1, gk2, z, one_u)

    boost = alpha >= f1
    alpha_b = jnp.where(boost, alpha, alpha + f1)
    d = alpha_b - jnp.float32(1.0 / 3.0)
    c = jnp.float32(1.0 / 3.0) / sqrt(d)

    def reject(X, V, U):
        c1 = U >= f1 - jnp.float32(0.0331) * (X * X)
        c2 = log(U) >= X * jnp.float32(0.5) + d * ((f1 - V) + log(V))
        return c1 & c2

    def draw_v(xk1, xk2):
        sk1, sk2 = _tf2x32(xk1, xk2, z, one_u)
        xn = normal(sk1, sk2)
        return xn, f1 + xn * c

    def inner(xk1, xk2):
        x, v = draw_v(xk1, xk2)

        def inner_cond(ic):
            return jnp.any(ic[3] != 0)

        def inner_body(ic):
            xk1, xk2, x, acti = ic
            act = acti != 0
            nxk1, nxk2 = _tf2x32(xk1, xk2, z, z)
            xk1 = jnp.where(act, nxk1, xk1)
            xk2 = jnp.where(act, nxk2, xk2)
            xn, vn = draw_v(xk1, xk2)
            x = jnp.where(act, xn, x)
            nact = act & (vn <= 0)
            return xk1, xk2, x, jnp.where(nact, np.int32(1), np.int32(0))

        _, _, x, _ = jax.lax.while_loop(
            inner_cond, inner_body,
            (xk1, xk2, x, jnp.where(v <= 0, np.int32(1), np.int32(0))))
        return x

    def one_round(k1, k2):
        xk1, xk2 = _tf2x32(k1, k2, z, one_u)
        uk1, uk2 = _tf2x32(k1, k2, z, two_u)
        x = inner(xk1, xk2)
        Xn = x * x
        Vn = x * c + f1
        Vn = (Vn * Vn) * Vn
        Un = _uniform01(uk1, uk2)
        return Xn, Vn, Un

    X1, V1, U1 = one_round(a1, a2)
    m1 = reject(X1, V1, U1)

    def outer_cond(carry):
        return jnp.any(carry[3] != 0)

    def outer_body(carry):
        k1, k2, V, mi = carry
        m = mi != 0
        nk1, nk2 = _tf2x32(k1, k2, z, z)
        k1 = jnp.where(m, nk1, k1)
        k2 = jnp.where(m, nk2, k2)
        Xn, Vn, Un = one_round(k1, k2)
        V = jnp.where(m, Vn, V)
        nm = m & reject(Xn, Vn, Un)
        return k1, k2, V, jnp.where(nm, np.int32(1), np.int32(0))

    _, _, V, _ = jax.lax.while_loop(
        outer_cond, outer_body,
        (a1, a2, V1, jnp.where(m1, np.int32(1), np.int32(0))))

    u_exp = _uniform01(s1, s2)
    log_samples = log1p(-u_exp)
    log_boost = jnp.where(boost | (log_samples == 0), jnp.float32(0.0),
                          log_samples * (f1 / alpha))
    return (log(d) + log(V)) + log_boost


_LANES = 128
_BLOCK_ROWS = 64
_SUB_ROWS = 8


def _ts_kernel(n_total, block_elems):
    def body(alpha_ref, beta_ref, scores_ref, bmax_ref, bidx_ref):
        g = pl.program_id(0)
        a = alpha_ref[...]
        b = beta_ref[...]
        shape = a.shape

        base = (g * np.int32(block_elems)).astype(jnp.int32)
        row_i = jax.lax.broadcasted_iota(jnp.int32, shape, 0)
        col_i = jax.lax.broadcasted_iota(jnp.int32, shape, 1)
        lin_i = base + row_i * np.int32(_LANES) + col_i
        lin_u = lin_i.astype(jnp.uint32)

        zu = jnp.zeros_like(lin_u)
        ga1, ga2 = _tf2x32(jnp.full(shape, np.uint32(_KA1)),
                           jnp.full(shape, np.uint32(_KA2)), zu, lin_u)
        gb1, gb2 = _tf2x32(jnp.full(shape, np.uint32(_KB1)),
                           jnp.full(shape, np.uint32(_KB2)), zu, lin_u)

        lga = _loggamma(ga1, ga2, a)
        lgb = _loggamma(gb1, gb2, b)
        log_max = jnp.maximum(lga, lgb)
        sa = jnp.exp(lga - log_max)
        sb = jnp.exp(lgb - log_max)
        scores = sa / (sa + sb)
        scores_ref[...] = scores

        valid = lin_i < np.int32(n_total)
        sc = jnp.where(valid, scores, jnp.float32(-1.0))
        blk_max = jnp.max(sc)
        blk_idx = jnp.min(jnp.where(sc == blk_max, lin_i, np.int32(2**31 - 1)))
        bmax_ref[0, 0, 0] = blk_max
        bidx_ref[0, 0, 0] = blk_idx

    return body


def _argmax_combine(grid_n):
    def body(bmax_ref, bidx_ref, smax_ref, sidx_ref, action_ref):
        def step(g, carry):
            bv, bi = carry
            mv = bmax_ref[g, 0, 0]
            mi = bidx_ref[g, 0, 0]
            take = mv > bv
            return (jnp.where(take, mv, bv), jnp.where(take, mi, bi))

        tc_v, tc_i = jax.lax.fori_loop(
            0, grid_n, step, (jnp.float32(-2.0), jnp.int32(0)))

        sm = smax_ref[...]
        si = sidx_ref[...]
        sc_v = jnp.max(sm)
        sc_i = jnp.min(jnp.where(sm == sc_v, si, np.int32(2**31 - 1)))
        take_sc = sc_v > tc_v
        action_ref[0] = jnp.where(take_sc, sc_i, tc_i)

    return body


_SC_W = 32
_SC_L = 16
_SC_CHUNK = _SC_W * _SC_L


_SC_ROUNDS = 3
_SC_UNROLL = 2


def _loggamma_fixed(gk1, gk2, alpha):
    z = jnp.zeros_like(gk1)
    one_u = z + np.uint32(1)
    two_u = z + np.uint32(2)
    f1 = jnp.float32(1.0)

    a1, a2 = _tf2x32(gk1, gk2, z, z)
    s1, s2 = _tf2x32(gk1, gk2, z, one_u)

    boost = alpha >= f1
    alpha_b = jnp.where(boost, alpha, alpha + f1)
    d = alpha_b - jnp.float32(1.0 / 3.0)
    c = jnp.float32(1.0 / 3.0) / _soft_sqrt(d)

    def reject(X, V, U):
        c1 = U >= f1 - jnp.float32(0.0331) * (X * X)
        c2 = _soft_log(U) >= X * jnp.float32(0.5) + d * ((f1 - V) +
                                                         _soft_log(V))
        return c1 & c2

    def one_round(k1, k2):
        xk1, xk2 = _tf2x32(k1, k2, z, one_u)
        uk1, uk2 = _tf2x32(k1, k2, z, two_u)
        sk1, sk2 = _tf2x32(xk1, xk2, z, one_u)
        x = _normal_soft(sk1, sk2)
        Xn = x * x
        Vn = x * c + f1
        Vn = (Vn * Vn) * Vn
        Un = _uniform01(uk1, uk2)
        return Xn, Vn, Un

    X1, V1, U1 = one_round(a1, a2)
    m1 = reject(X1, V1, U1)

    def round_body(_, carry):
        k1, k2, V, mi = carry
        m = mi != 0
        nk1, nk2 = _tf2x32(k1, k2, z, z)
        k1 = jnp.where(m, nk1, k1)
        k2 = jnp.where(m, nk2, k2)
        Xn, Vn, Un = one_round(k1, k2)
        V = jnp.where(m, Vn, V)
        nm = m & reject(Xn, Vn, Un)
        return k1, k2, V, jnp.where(nm, np.int32(1), np.int32(0))

    _, _, V, _ = jax.lax.fori_loop(
        0, _SC_ROUNDS - 1, round_body,
        (a1, a2, V1, jnp.where(m1, np.int32(1), np.int32(0))))

    u_exp = _uniform01(s1, s2)
    log_samples = _soft_log1p(-u_exp)
    log_boost = jnp.where(boost | (log_samples == 0), jnp.float32(0.0),
                          log_samples * (f1 / alpha))
    return (_soft_log(d) + _soft_log(V)) + log_boost


def _sc_body(n_tc, per_tile):

    def body(alpha_hbm, beta_hbm, scores_hbm, smax_hbm, sidx_hbm,
             a_v, b_v, s_v, stv_v, sti_v, mx_v, mi_v):
        cid = jax.lax.axis_index("c")
        sid = jax.lax.axis_index("s")
        wid = sid * 2 + cid
        base = wid * per_tile
        pltpu.sync_copy(alpha_hbm.at[pl.ds(base, per_tile)], a_v)
        pltpu.sync_copy(beta_hbm.at[pl.ds(base, per_tile)], b_v)

        lane = jax.lax.iota(jnp.int32, _SC_L)
        lane_u = jax.lax.iota(jnp.uint32, _SC_L)
        base_u = base.astype(jnp.uint32)

        def step(g, carry):
            bm, bi = carry
            for j in range(_SC_UNROLL):
                off = (g * _SC_UNROLL + j) * _SC_L
                a = a_v[pl.ds(off, _SC_L)]
                b = b_v[pl.ds(off, _SC_L)]
                lin = (np.int32(n_tc) + base + off) + lane
                lin_u = base_u + np.uint32(n_tc) + jnp.uint32(off) + lane_u
                zu = jnp.zeros_like(lin_u)
                ka1 = zu + np.uint32(_KA1)
                ka2 = zu + np.uint32(_KA2)
                kb1 = zu + np.uint32(_KB1)
                kb2 = zu + np.uint32(_KB2)
                ga1, ga2 = _tf2x32(ka1, ka2, zu, lin_u)
                gb1, gb2 = _tf2x32(kb1, kb2, zu, lin_u)
                lga = _loggamma_fixed(ga1, ga2, a)
                lgb = _loggamma_fixed(gb1, gb2, b)
                log_max = jnp.maximum(lga, lgb)
                sa = jnp.exp(lga - log_max)
                sb = jnp.exp(lgb - log_max)
                sc = sa / (sa + sb)
                s_v[pl.ds(off, _SC_L)] = sc
                upd = sc > bm
                bm = jnp.where(upd, sc, bm)
                bi = jnp.where(upd, lin, bi)
            return bm, bi

        bm0 = jnp.full((_SC_L,), -2.0, jnp.float32)
        bi0 = jnp.zeros((_SC_L,), jnp.int32)
        bm, bi = jax.lax.fori_loop(
            0, per_tile // (_SC_L * _SC_UNROLL), step, (bm0, bi0))

        stv_v[...] = bm
        sti_v[...] = bi
        pltpu.sync_copy(s_v, scores_hbm.at[pl.ds(base, per_tile)])
        pltpu.sync_copy(stv_v, smax_hbm.at[wid])
        pltpu.sync_copy(sti_v, sidx_hbm.at[wid])

    return body


def _sc_sample(alpha_sc, beta_sc, n_tc):
    import functools
    from jax.experimental.pallas import tpu_sc as plsc
    n_sc = alpha_sc.shape[0]
    per_tile = n_sc // _SC_W
    mesh = plsc.VectorSubcoreMesh(core_axis_name="c", subcore_axis_name="s")
    k = functools.partial(
        pl.kernel,
        mesh=mesh,
        out_type=[
            jax.ShapeDtypeStruct((n_sc,), jnp.float32),
            jax.ShapeDtypeStruct((_SC_W, _SC_L), jnp.float32),
            jax.ShapeDtypeStruct((_SC_W, _SC_L), jnp.int32),
        ],
        scratch_types=[
            pltpu.VMEM((per_tile,), jnp.float32),
            pltpu.VMEM((per_tile,), jnp.float32),
            pltpu.VMEM((per_tile,), jnp.float32),
            pltpu.VMEM((_SC_L,), jnp.float32),
            pltpu.VMEM((_SC_L,), jnp.int32),
            pltpu.VMEM((_SC_L,), jnp.float32),
            pltpu.VMEM((_SC_L,), jnp.int32),
        ],
    )(_sc_body(n_tc, per_tile))
    return k(alpha_sc, beta_sc)


_SC_FRAC_NUM = 1
_SC_FRAC_DEN = 12


def kernel(alpha, beta):
    n = alpha.shape[0]
    n_sc = ((n * _SC_FRAC_NUM) // _SC_FRAC_DEN) // _SC_CHUNK * _SC_CHUNK
    n_tc = n - n_sc
    return _kernel_split(alpha, beta, n, n_tc, n_sc)


def _kernel_split(alpha, beta, n, n_tc, n_sc):
    rows = -(-n_tc // _LANES)
    rows_pad = -(-rows // _BLOCK_ROWS) * _BLOCK_ROWS
    total = rows_pad * _LANES
    grid_n = rows_pad // _BLOCK_ROWS
    block_elems = _BLOCK_ROWS * _LANES

    a2d = jnp.concatenate(
        [alpha[:n_tc],
         jnp.ones((total - n_tc,), jnp.float32)]).reshape(rows_pad, _LANES)
    b2d = jnp.concatenate(
        [beta[:n_tc],
         jnp.ones((total - n_tc,), jnp.float32)]).reshape(rows_pad, _LANES)

    scores2d, bmax, bidx = pl.pallas_call(
        _ts_kernel(n_tc, block_elems),
        grid=(grid_n,),
        in_specs=[
            pl.BlockSpec((_BLOCK_ROWS, _LANES), lambda g: (g, 0)),
            pl.BlockSpec((_BLOCK_ROWS, _LANES), lambda g: (g, 0)),
        ],
        out_specs=[
            pl.BlockSpec((_BLOCK_ROWS, _LANES), lambda g: (g, 0)),
            pl.BlockSpec((1, 1, 1), lambda g: (g, 0, 0), memory_space=pltpu.SMEM),
            pl.BlockSpec((1, 1, 1), lambda g: (g, 0, 0), memory_space=pltpu.SMEM),
        ],
        out_shape=[
            jax.ShapeDtypeStruct((rows_pad, _LANES), jnp.float32),
            jax.ShapeDtypeStruct((grid_n, 1, 1), jnp.float32),
            jax.ShapeDtypeStruct((grid_n, 1, 1), jnp.int32),
        ],
        compiler_params=pltpu.CompilerParams(
            dimension_semantics=("parallel",)),
    )(a2d, b2d)

    if n_sc:
        sc_scores, smax, sidx = _sc_sample(alpha[n_tc:], beta[n_tc:], n_tc)
    else:
        sc_scores = jnp.zeros((0,), jnp.float32)
        smax = jnp.full((_SC_W, _SC_L), -2.0, jnp.float32)
        sidx = jnp.zeros((_SC_W, _SC_L), jnp.int32)

    action1 = pl.pallas_call(
        _argmax_combine(grid_n),
        in_specs=[
            pl.BlockSpec(memory_space=pltpu.SMEM),
            pl.BlockSpec(memory_space=pltpu.SMEM),
            pl.BlockSpec((4, _SC_W * _SC_L // 4), lambda: (0, 0)),
            pl.BlockSpec((4, _SC_W * _SC_L // 4), lambda: (0, 0)),
        ],
        out_specs=pl.BlockSpec(memory_space=pltpu.SMEM),
        out_shape=jax.ShapeDtypeStruct((1,), jnp.int32),
    )(bmax, bidx,
      smax.reshape(4, _SC_W * _SC_L // 4),
      sidx.reshape(4, _SC_W * _SC_L // 4))

    scores = jnp.concatenate([scores2d.reshape(-1)[:n_tc], sc_scores])
    action = action1[0]
    return (action, scores)
